# baseline (speedup 1.0000x reference)
import functools

import jax
import jax.numpy as jnp
from jax import lax
from jax.experimental import pallas as pl
from jax.experimental.pallas import tpu as pltpu
from jax.experimental.pallas import tpu_sc as plsc

VOCAB = 128000
D_MODEL = 4096
NTOK = 16384

_info = plsc.get_sparse_core_info()
NC, NS = _info.num_cores, _info.num_subcores
NW = NC * NS
K = 8
NBUF = 3

NTOK_TC = 4096
NTOK_SC = NTOK - NTOK_TC
R_TC = 16


def _make_sc_gather(ntok):
    tpw = ntok // NW
    nchunks = tpw // K
    assert nchunks % NBUF == 0

    @functools.partial(
        pl.kernel,
        mesh=plsc.VectorSubcoreMesh(core_axis_name="c", subcore_axis_name="s"),
        out_type=jax.ShapeDtypeStruct((ntok, D_MODEL), jnp.float32),
        scratch_types=[
            pltpu.VMEM((tpw,), jnp.int32),
            pltpu.VMEM((K, D_MODEL), jnp.float32),
            pltpu.VMEM((K, D_MODEL), jnp.float32),
            pltpu.VMEM((K, D_MODEL), jnp.float32),
            pltpu.SemaphoreType.DMA,
            pltpu.SemaphoreType.DMA,
            pltpu.SemaphoreType.DMA,
        ],
    )
    def sc_gather(ids_hbm, table_hbm, out_hbm, idx_v, buf0, buf1, buf2, sem0, sem1, sem2):
        wid = lax.axis_index("s") * NC + lax.axis_index("c")
        base = wid * tpw
        pltpu.sync_copy(ids_hbm.at[pl.ds(base, tpw)], idx_v)
        bufs = (buf0, buf1, buf2)
        sems = (sem0, sem1, sem2)

        def gather(c, j):
            pltpu.async_copy(table_hbm.at[idx_v.at[pl.ds(c * K, K)]], bufs[j], sems[j])

        def wait(j):
            pltpu.make_async_copy(table_hbm.at[pl.ds(0, K)], bufs[j], sems[j]).wait()

        def write_out(c, j):
            pltpu.sync_copy(bufs[j], out_hbm.at[pl.ds(base + c * K, K)])

        for j in range(NBUF):
            gather(j, j)

        def ring_body(g, carry):
            for j in range(NBUF):
                c = NBUF * g + j
                wait(j)
                write_out(c, j)

                @pl.when(c + NBUF < nchunks)
                def _():
                    gather(c + NBUF, j)

            return carry

        lax.fori_loop(0, nchunks // NBUF, ring_body, 0)

    return sc_gather


def _tc_body(ids_ref, table_ref, out_ref, sem):
    i = pl.program_id(0)
    for r in range(R_TC):
        row = ids_ref[i * R_TC + r]
        pltpu.make_async_copy(
            table_ref.at[pl.ds(row, 1)], out_ref.at[pl.ds(r, 1)], sem
        ).start()
    for r in range(R_TC):
        pltpu.make_async_copy(
            table_ref.at[pl.ds(0, 1)], out_ref.at[pl.ds(r, 1)], sem
        ).wait()


def _tc_gather(ids, table):
    grid_spec = pltpu.PrefetchScalarGridSpec(
        num_scalar_prefetch=1,
        grid=(NTOK_TC // R_TC,),
        in_specs=[pl.BlockSpec(memory_space=pltpu.MemorySpace.HBM)],
        out_specs=pl.BlockSpec((R_TC, D_MODEL), lambda i, ids: (i, 0)),
        scratch_shapes=[pltpu.SemaphoreType.DMA],
    )
    return pl.pallas_call(
        _tc_body,
        grid_spec=grid_spec,
        out_shape=jax.ShapeDtypeStruct((NTOK_TC, D_MODEL), jnp.float32),
    )(ids, table)


_sc_gather = _make_sc_gather(NTOK_SC)


def kernel(input_ids, embed_weight):
    ids_flat = input_ids.reshape(NTOK).astype(jnp.int32)
    out_sc = _sc_gather(ids_flat[:NTOK_SC], embed_weight)
    out_tc = _tc_gather(ids_flat[NTOK_SC:], embed_weight)
    out = jnp.concatenate([out_sc, out_tc], axis=0)
    return out.reshape(input_ids.shape[0], input_ids.shape[1], D_MODEL)


# speedup vs baseline: 2.6407x; 2.6407x over previous
import functools

import jax
import jax.numpy as jnp
from jax import lax
from jax.experimental import pallas as pl
from jax.experimental.pallas import tpu as pltpu
from jax.experimental.pallas import tpu_sc as plsc

VOCAB = 128000
D_MODEL = 4096
NTOK = 16384

_info = plsc.get_sparse_core_info()
NC, NS = _info.num_cores, _info.num_subcores
NW = NC * NS
TPW = NTOK // NW
K = 8
ROUND = 3 * K
NROUNDS = TPW // ROUND


@functools.partial(
    pl.kernel,
    mesh=plsc.VectorSubcoreMesh(core_axis_name="c", subcore_axis_name="s"),
    out_type=jax.ShapeDtypeStruct((NTOK, D_MODEL), jnp.float32),
    scratch_types=[
        pltpu.VMEM((TPW,), jnp.int32),
        pltpu.VMEM((ROUND, D_MODEL), jnp.float32),
        pltpu.SemaphoreType.DMA,
        pltpu.SemaphoreType.DMA,
        pltpu.SemaphoreType.DMA,
    ],
)
def _emb_lookup(ids_hbm, table_hbm, out_hbm, idx_v, buf, sem0, sem1, sem2):
    wid = lax.axis_index("s") * NC + lax.axis_index("c")
    base = wid * TPW
    pltpu.sync_copy(ids_hbm.at[pl.ds(base, TPW)], idx_v)
    sems = (sem0, sem1, sem2)

    def gather(tok, j):
        pltpu.async_copy(
            table_hbm.at[idx_v.at[pl.ds(tok, K)]], buf.at[pl.ds(j * K, K)], sems[j]
        )

    def wait(j):
        pltpu.make_async_copy(
            table_hbm.at[pl.ds(0, K)], buf.at[pl.ds(j * K, K)], sems[j]
        ).wait()

    def write_out(row0, nrows, tok):
        pltpu.sync_copy(
            buf.at[pl.ds(row0, nrows)], out_hbm.at[pl.ds(base + tok, nrows)]
        )

    gather(0, 0)
    gather(K, 1)
    gather(2 * K, 2)

    def round_body(r, carry):
        t = ROUND * r
        wait(0)
        wait(1)
        write_out(0, 2 * K, t)
        gather(t + ROUND, 0)

        @pl.when(r + 1 < NROUNDS)
        def _():
            gather(t + ROUND + K, 1)

        wait(2)
        write_out(2 * K, K, t + 2 * K)

        @pl.when(r + 1 < NROUNDS)
        def _():
            gather(t + ROUND + 2 * K, 2)

        return carry

    lax.fori_loop(0, NROUNDS, round_body, 0)
    wait(0)
    write_out(0, K, TPW - K)


def kernel(input_ids, embed_weight):
    ids_flat = input_ids.reshape(NTOK).astype(jnp.int32)
    out = _emb_lookup(ids_flat, embed_weight)
    return out.reshape(input_ids.shape[0], input_ids.shape[1], D_MODEL)
